# Initial kernel scaffold; baseline (speedup 1.0000x reference)
import jax
import jax.numpy as jnp
from jax.experimental import pallas as pl


def kernel(x, edge_index, edge_attr, batch, emb_node, W_edge, b_edge, W1, b1, W2, b2, Wmu, bmu, Wlv, blv, Wa1, ba1, Wa2, ba2, Wc1, bc1, Wc2, bc2, Wc3, bc3, Wn1, bn1, Wn2, bn2):
    raise NotImplementedError("write your pallas kernel here")


# speedup vs baseline: 4.1035x; 4.1035x over previous
import functools

import jax
import jax.numpy as jnp
from jax import lax
from jax.experimental import pallas as pl
from jax.experimental.pallas import tpu as pltpu
from jax.experimental.pallas import tpu_sc as plsc

N = 10000
E = 320000
G = 64
IN_DIM = 128
HID = 256
LAT = 128
EDGE_DIM = 7

NC = 2
NS = 16
BLK = 80
HALF = N // NC
HPAD = 5120
NPAD = 10240

def _edge_stats_body(emb2, src, dst, zeros_hbm, ones_hbm,
                     out_nef, out_cnt, acc, ebuf, obuf, sbuf, dbuf):
    c = lax.axis_index("c")
    s = lax.axis_index("s")
    zrows = NPAD // NS
    pltpu.sync_copy(zeros_hbm.at[pl.ds(s * zrows, zrows)],
                    acc.at[pl.ds(s * zrows, zrows)])
    pltpu.sync_copy(ones_hbm, obuf)
    plsc.subcore_barrier()

    base = s * (E // NS)

    def body(j, _):
        off = base + j * BLK
        pltpu.sync_copy(src.at[pl.ds(off, BLK)], sbuf)
        pltpu.sync_copy(dst.at[pl.ds(off, BLK)], dbuf)
        pltpu.sync_copy(emb2.at[c, pl.ds(off, BLK)], ebuf)
        pltpu.sync_copy(ebuf, acc.at[sbuf], add=True)
        pltpu.sync_copy(ebuf, acc.at[dbuf], add=True)
        return 0

    lax.fori_loop(0, E // NS // BLK, body, 0)
    plsc.subcore_barrier()
    pltpu.sync_copy(acc.at[pl.ds(s * zrows, zrows)],
                    out_nef.at[c, pl.ds(s * zrows, zrows)])
    plsc.subcore_barrier()
    pltpu.sync_copy(zeros_hbm.at[pl.ds(s * zrows, zrows)],
                    acc.at[pl.ds(s * zrows, zrows)])
    plsc.subcore_barrier()

    cbase = (c * NS + s) * (E // (NC * NS))

    def body_cnt(j, _):
        off = cbase + j * BLK
        pltpu.sync_copy(dst.at[pl.ds(off, BLK)], dbuf)
        pltpu.sync_copy(obuf, acc.at[dbuf], add=True)
        return 0

    lax.fori_loop(0, E // (NC * NS) // BLK, body_cnt, 0)
    plsc.subcore_barrier()
    pltpu.sync_copy(acc.at[pl.ds(s * zrows, zrows)],
                    out_cnt.at[c, pl.ds(s * zrows, zrows)])


@functools.lru_cache(maxsize=None)
def _sc_kernels():
    mesh = plsc.VectorSubcoreMesh(core_axis_name="c", subcore_axis_name="s",
                                  num_cores=NC, num_subcores=NS)
    edge_stats = pl.kernel(
        _edge_stats_body,
        out_type=[jax.ShapeDtypeStruct((NC, NPAD, HID // 2), jnp.float32),
                  jax.ShapeDtypeStruct((NC, NPAD, HID // 2), jnp.float32)],
        mesh=mesh,
        scratch_types=[
            pltpu.VMEM_SHARED((NPAD, HID // 2), jnp.float32),
            pltpu.VMEM((BLK, HID // 2), jnp.float32),
            pltpu.VMEM((BLK, HID // 2), jnp.float32),
            pltpu.VMEM((BLK,), jnp.int32),
            pltpu.VMEM((BLK,), jnp.int32),
        ],
    )
    agg = pl.kernel(
        _agg_body,
        out_type=jax.ShapeDtypeStruct((NC, 2, HPAD, HID // 2), jnp.float32),
        mesh=mesh,
        scratch_types=[
            pltpu.VMEM_SHARED((HPAD, HID // 2), jnp.float32),
            pltpu.VMEM_SHARED((HPAD, HID // 2), jnp.float32),
            pltpu.VMEM((BLK, HID // 2), jnp.float32),
            pltpu.VMEM((BLK, HID // 2), jnp.float32),
            pltpu.VMEM((BLK,), jnp.int32),
            pltpu.VMEM((BLK,), jnp.int32),
            pltpu.VMEM((BLK,), jnp.int32),
            pltpu.SemaphoreType.DMA,
        ],
    )
    return edge_stats, agg


def _agg_body(ht_a, ht_b, src, dst, zeros_hbm, out,
              acc_a, acc_b, rows_a, rows_b, sbuf, dbuf, lbuf, sem):
    c = lax.axis_index("c")
    s = lax.axis_index("s")
    zrows = HPAD // NS
    pltpu.sync_copy(zeros_hbm.at[pl.ds(s * zrows, zrows)],
                    acc_a.at[pl.ds(s * zrows, zrows)])
    pltpu.sync_copy(zeros_hbm.at[pl.ds(s * zrows, zrows)],
                    acc_b.at[pl.ds(s * zrows, zrows)])
    plsc.subcore_barrier()

    cbase = c * HALF
    per_worker = E // NS
    base = s * per_worker

    def body(j, _):
        off = base + j * BLK
        pltpu.sync_copy(src.at[pl.ds(off, BLK)], sbuf)
        pltpu.sync_copy(dst.at[pl.ds(off, BLK)], dbuf)
        for i in range(BLK // 16):
            d = dbuf[pl.ds(i * 16, 16)]
            ok = (d >= cbase) & (d < cbase + HALF)
            lbuf[pl.ds(i * 16, 16)] = jnp.where(ok, d - cbase,
                                                HALF + (d & 7))
        pltpu.async_copy(ht_a.at[sbuf], rows_a, sem).wait()
        pltpu.sync_copy(rows_a, acc_a.at[lbuf], add=True)
        pltpu.async_copy(ht_b.at[sbuf], rows_b, sem).wait()
        pltpu.sync_copy(rows_b, acc_b.at[lbuf], add=True)
        return 0

    lax.fori_loop(0, per_worker // BLK, body, 0)
    plsc.subcore_barrier()
    pltpu.sync_copy(acc_a.at[pl.ds(s * zrows, zrows)],
                    out.at[c, 0, pl.ds(s * zrows, zrows)])
    pltpu.sync_copy(acc_b.at[pl.ds(s * zrows, zrows)],
                    out.at[c, 1, pl.ds(s * zrows, zrows)])


EBLK = 3200


def _tc0_body(ea8_ref, we8_ref, emb_ref):
    emb_ref[0] = jnp.dot(ea8_ref[...], we8_ref[...],
                         preferred_element_type=jnp.float32)


def _split(p, a_ref, b_ref):
    a_ref[...] = p[:, 0:HID // 2]
    b_ref[...] = p[:, HID // 2:HID]


def _assemble(agg_ref):
    top = jnp.concatenate([agg_ref[0, 0, 0:HALF, :], agg_ref[0, 1, 0:HALF, :]], 1)
    bot = jnp.concatenate([agg_ref[1, 0, 0:HALF, :], agg_ref[1, 1, 0:HALF, :]], 1)
    return jnp.concatenate([top, bot], 0)


def _tc1_body(nef_ref, cnt_ref, emb_ref, w1a_ref, w1b_ref,
              ht1a_ref, ht1b_ref, dinv_ref):
    nef = jnp.concatenate([nef_ref[0, 0:N, :], nef_ref[1, 0:N, :]], 1)
    indeg = cnt_ref[0, 0:N, 0:1] + cnt_ref[1, 0:N, 0:1]
    dinv = lax.rsqrt(indeg + 1.0)
    row = jnp.dot(emb_ref[...], w1a_ref[...],
                  preferred_element_type=jnp.float32)
    p1 = jnp.dot(nef, w1b_ref[...], preferred_element_type=jnp.float32) + row
    _split(p1 * dinv, ht1a_ref, ht1b_ref)
    dinv_ref[...] = dinv


def _tc2_body(agg_ref, hta_ref, htb_ref, dinv_ref, b_ref, w_ref,
              outa_ref, outb_ref):
    dinv = dinv_ref[...]
    ht = jnp.concatenate([hta_ref[...], htb_ref[...]], 1)
    h = jnp.maximum((_assemble(agg_ref) + ht) * dinv + b_ref[...], 0.0)
    _split(jnp.dot(h, w_ref[...],
                   preferred_element_type=jnp.float32) * dinv,
           outa_ref, outb_ref)


def _tc3_body(agg_ref, hta_ref, htb_ref, dinv_ref, b_ref,
              outa_ref, outb_ref):
    dinv = dinv_ref[...]
    ht = jnp.concatenate([hta_ref[...], htb_ref[...]], 1)
    h = jnp.maximum((_assemble(agg_ref) + ht) * dinv + b_ref[...], 0.0)
    _split(h * dinv, outa_ref, outb_ref)


def _tc4_body(agg_ref, hta_ref, htb_ref, dinv_ref, batch_ref,
              wmu_ref, bmu_ref, wlv_ref, blv_ref,
              wa1_ref, ba1_ref, wa2_ref, ba2_ref,
              wc1_ref, bc1_ref, wc2_ref, bc2_ref, wc3_ref, bc3_ref,
              wn1_ref, bn1_ref, wn2_ref, bn2_ref,
              pred_ref, mu_ref, lv_ref, nc_ref):
    ht = jnp.concatenate([hta_ref[...], htb_ref[...]], 1)
    a = (_assemble(agg_ref) + ht) * dinv_ref[...]
    mu = jnp.dot(a, wmu_ref[...], preferred_element_type=jnp.float32) \
        + bmu_ref[...]
    lv = jnp.dot(a, wlv_ref[...], preferred_element_type=jnp.float32) \
        + blv_ref[...]
    mu_ref[...] = mu
    lv_ref[...] = lv
    t = jnp.tanh(jnp.dot(mu, wa1_ref[...],
                         preferred_element_type=jnp.float32) + ba1_ref[...])
    attl = jnp.dot(t, wa2_ref[...],
                   preferred_element_type=jnp.float32) + ba2_ref[...]
    m = jnp.max(attl)
    p = jnp.exp(attl - m)
    att = p / jnp.sum(p)
    onehot = (batch_ref[...] ==
              lax.broadcasted_iota(jnp.int32, (G, 1), 0)).astype(jnp.float32)
    za = mu * att
    gr = jnp.dot(onehot, za, preferred_element_type=jnp.float32)
    cnts = jnp.sum(onehot, axis=1, keepdims=True)
    hc = jnp.maximum(jnp.dot(gr, wc1_ref[...],
                             preferred_element_type=jnp.float32)
                     + bc1_ref[...], 0.0)
    hc = jnp.maximum(jnp.dot(hc, wc2_ref[...],
                             preferred_element_type=jnp.float32)
                     + bc2_ref[...], 0.0)
    pred_ref[...] = jnp.dot(hc, wc3_ref[...],
                            preferred_element_type=jnp.float32) + bc3_ref[...]
    sumz = jnp.dot(onehot, mu, preferred_element_type=jnp.float32)
    meanp = sumz / jnp.maximum(cnts, 1.0)
    hn = jnp.maximum(jnp.dot(meanp, wn1_ref[...],
                             preferred_element_type=jnp.float32)
                     + bn1_ref[...], 0.0)
    nl = jnp.dot(hn, wn2_ref[...], preferred_element_type=jnp.float32) \
        + bn2_ref[...]
    nc_ref[...] = 1.0 / (1.0 + jnp.exp(-nl))


def _tc_call(body, out_shapes, *args):
    return pl.pallas_call(body, out_shape=out_shapes)(*args)


def kernel(x, edge_index, edge_attr, batch, emb_node, W_edge, b_edge,
           W1, b1, W2, b2, Wmu, bmu, Wlv, blv, Wa1, ba1, Wa2, ba2,
           Wc1, bc1, Wc2, bc2, Wc3, bc3, Wn1, bn1, Wn2, bn2):
    src = edge_index[0]
    dst = edge_index[1]
    ea8 = jnp.concatenate(
        [edge_attr, jnp.ones((E, 1), jnp.float32)], axis=1)
    zeros_n = jnp.zeros((NPAD, HID // 2), jnp.float32)
    ones_b = jnp.ones((BLK, HID // 2), jnp.float32)

    we8 = jnp.concatenate([W_edge, b_edge[None, :]], axis=0)
    hw = HID // 2
    emb2 = pl.pallas_call(
        _tc0_body,
        grid=(2, E // EBLK),
        in_specs=[pl.BlockSpec((EBLK, 8), lambda j, i: (i, 0)),
                  pl.BlockSpec((8, hw), lambda j, i: (0, j))],
        out_specs=pl.BlockSpec((1, EBLK, hw), lambda j, i: (j, i, 0)),
        out_shape=jax.ShapeDtypeStruct((2, E, hw), jnp.float32),
    )(ea8, we8)

    edge_stats, agg = _sc_kernels()
    nef_h, cnt_h = edge_stats(emb2, src, dst, zeros_n, ones_b)

    w1a = W1[0:IN_DIM, :]
    w1b = W1[IN_DIM:, :]
    half_t = jax.ShapeDtypeStruct((N, HID // 2), jnp.float32)
    ht1a, ht1b, dinv = _tc_call(
        _tc1_body,
        [half_t, half_t, jax.ShapeDtypeStruct((N, 1), jnp.float32)],
        nef_h, cnt_h, emb_node, w1a, w1b)

    acc1 = agg(ht1a, ht1b, src, dst, zeros_n)
    ht2a, ht2b = _tc_call(
        _tc2_body, [half_t, half_t],
        acc1, ht1a, ht1b, dinv, b1[None, :], W2)

    acc2 = agg(ht2a, ht2b, src, dst, zeros_n)
    ht3a, ht3b = _tc_call(
        _tc3_body, [half_t, half_t],
        acc2, ht2a, ht2b, dinv, b2[None, :])

    acc3 = agg(ht3a, ht3b, src, dst, zeros_n)
    batch2 = batch.reshape(1, N)
    pred, mu, lv, nc = _tc_call(
        _tc4_body,
        [jax.ShapeDtypeStruct((G, 6), jnp.float32),
         jax.ShapeDtypeStruct((N, LAT), jnp.float32),
         jax.ShapeDtypeStruct((N, LAT), jnp.float32),
         jax.ShapeDtypeStruct((G, 1), jnp.float32)],
        acc3, ht3a, ht3b, dinv, batch2,
        Wmu, bmu[None, :], Wlv, blv[None, :],
        Wa1, ba1[None, :], Wa2, ba2[None, :],
        Wc1, bc1[None, :], Wc2, bc2[None, :], Wc3, bc3[None, :],
        Wn1, bn1[None, :], Wn2, bn2[None, :])
    return (pred, mu, lv, nc)
